# baseline (speedup 1.0000x reference)
import functools

import jax
import jax.numpy as jnp
from jax import lax
from jax.experimental import pallas as pl
from jax.experimental.pallas import tpu as pltpu
from jax.experimental.pallas import tpu_sc as plsc

N = 10000
E = 320000
D = 128
C_CURV = 1.0
K_CURV = 1.0 / C_CURV
SQRTK = K_CURV ** 0.5
EPS = 1e-7
MIN_NORM = 1e-15

NC = 2
NS = 16
CK = 128
N_PAD = 10240
NSLICE = N_PAD // NS
ZROWS = 128
NCH = 80
EPT = NCH * CK
E_PAD = EPT * NC * NS
NCHT = NCH * NC * NS


def _prolog_body(h_ref, w_ref, wa_ref, wb_ref, hw_ref, aro_ref, acl_ref):
    h = h_ref[...]
    col = lax.broadcasted_iota(jnp.int32, (N, D), 1)
    is0 = col == 0
    h0 = h[:, 0:1]
    y = jnp.where(is0, 0.0, h)
    y_norm = jnp.maximum(jnp.sqrt(jnp.sum(y * y, axis=1, keepdims=True)), MIN_NORM)
    theta = jnp.maximum(h0 / SQRTK, 1.0 + EPS)
    arcosh = jnp.log(theta + jnp.sqrt(jnp.maximum(theta * theta - 1.0, MIN_NORM)))
    u = jnp.where(is0, 0.0, h * (SQRTK * arcosh / y_norm))
    hw = lax.dot_general(u, w_ref[...], (((1,), (1,)), ((), ())),
                         preferred_element_type=jnp.float32)
    hw_ref[...] = hw
    aro_ref[...] = jnp.sum(hw * wa_ref[...], axis=1, keepdims=True)
    acl_ref[...] = jnp.sum(hw * wb_ref[...], axis=1, keepdims=True)


def _edge_body(hw_hbm, aro_hbm, acl_hbm, pack_hbm, out_hbm, pk0, pk1, ar_v,
               ac_v, rows_v, srows_v, out_sh, sem, ssem):
    cid = lax.axis_index("c")
    sid = lax.axis_index("s")
    zv = jnp.zeros((16,), jnp.float32)

    def _zrow(r, _):
        for dd in range(8):
            rows_v[r, pl.ds(dd * 16, 16)] = zv
        return 0

    lax.fori_loop(0, ZROWS, _zrow, 0)
    base = sid * NSLICE
    for j in range(NSLICE // ZROWS):
        pltpu.sync_copy(rows_v, out_sh.at[pl.ds(base + j * ZROWS, ZROWS)])
    plsc.subcore_barrier()

    wid = cid * NS + sid
    dn = lax.GatherDimensionNumbers(offset_dims=(), collapsed_slice_dims=(0,),
                                    start_index_map=(0,))

    def _do_chunk(c, pack_v, first):
        cidx = wid * NCH + c
        pltpu.sync_copy(pack_hbm.at[cidx], pack_v)
        d1 = pltpu.async_copy(hw_hbm.at[pack_v.at[1]], rows_v, sem)
        d2 = pltpu.async_copy(aro_hbm.at[pack_v.at[0]], ar_v, sem)
        d3 = pltpu.async_copy(acl_hbm.at[pack_v.at[1]], ac_v, sem)
        d1.wait()
        d2.wait()
        d3.wait()
        if first:
            @pl.when(c > 0)
            def _():
                pltpu.make_async_copy(srows_v, out_sh.at[pack_v.at[0]],
                                      ssem).wait()
        else:
            pltpu.make_async_copy(srows_v, out_sh.at[pack_v.at[0]],
                                  ssem).wait()

        def _group(g, _):
            sl = pl.ds(g * 16, 16)
            x = ar_v[sl] + ac_v[sl] + plsc.bitcast(pack_v[2, sl], jnp.float32)
            att = plsc.bitcast(pack_v[3, sl], jnp.float32) / (1.0 + jnp.exp(-x))
            ge0 = g * 16
            for e in range(16):
                attb = lax.gather(att, jnp.full((16, 1), e, jnp.int32), dn,
                                  (1,),
                                  mode=lax.GatherScatterMode.PROMISE_IN_BOUNDS)
                for k in range(D // 16):
                    kd = pl.ds(k * 16, 16)
                    srows_v[ge0 + e, kd] = rows_v[ge0 + e, kd] * attb
            return 0

        lax.fori_loop(0, CK // 16, _group, 0)
        pltpu.async_copy(srows_v, out_sh.at[pack_v.at[0]], ssem, add=True)

    def _turn(t2, _):
        _do_chunk(t2 * 2, pk0, True)
        _do_chunk(t2 * 2 + 1, pk1, False)
        return 0

    lax.fori_loop(0, NCH // 2, _turn, 0)
    pltpu.make_async_copy(srows_v, out_sh.at[pk1.at[0]], ssem).wait()
    plsc.subcore_barrier()
    pltpu.sync_copy(out_sh.at[pl.ds(base, NSLICE)],
                    out_hbm.at[cid, pl.ds(base, NSLICE)])


def _epilog_body(o2_ref, g_ref, b_ref, out_ref):
    s = (o2_ref[0, 0:N] + o2_ref[1, 0:N]) * 0.01
    col = lax.broadcasted_iota(jnp.int32, (N, D), 1)
    is0 = col == 0
    dm1 = float(D - 1)
    s0 = s[:, 0:1]
    mu = (jnp.sum(s, axis=1, keepdims=True) - s0) / dm1
    dev = jnp.where(is0, 0.0, s - mu)
    var = jnp.sum(dev * dev, axis=1, keepdims=True) / dm1
    tn = dev / jnp.sqrt(var + 1e-5) * g_ref[...] + b_ref[...]
    o = jnp.where(is0, 0.0, tn)
    x_norm = jnp.maximum(jnp.sqrt(jnp.sum(o * o, axis=1, keepdims=True)), MIN_NORM)
    th = x_norm / SQRTK
    e = jnp.exp(th)
    ei = 1.0 / e
    ch = 0.5 * (e + ei)
    sh = 0.5 * (e - ei)
    res = jnp.where(is0, SQRTK * ch, SQRTK * sh * o / x_norm)
    yp = jnp.where(is0, 0.0, res)
    y_sq = jnp.sum(yp * yp, axis=1, keepdims=True)
    r0 = jnp.sqrt(jnp.maximum(K_CURV + y_sq, EPS))
    p = jnp.where(is0, 0.0, SQRTK * res / (r0 + SQRTK))
    p = p / (1.0 + jnp.exp(-p))
    sqn = jnp.sum(p * p, axis=1, keepdims=True)
    inv = SQRTK / (K_CURV - sqn)
    out_ref[...] = jnp.where(is0, (K_CURV + sqn) * inv, (2.0 * SQRTK) * p * inv)


@functools.lru_cache(maxsize=1)
def _make_edge_kernel():
    return pl.kernel(
        _edge_body,
        out_type=jax.ShapeDtypeStruct((NC, N_PAD, D), jnp.float32),
        mesh=plsc.VectorSubcoreMesh(core_axis_name="c", subcore_axis_name="s",
                                    num_cores=NC, num_subcores=NS),
        compiler_params=pltpu.CompilerParams(needs_layout_passes=False),
        scratch_types=[
            pltpu.VMEM((4, CK), jnp.int32),
            pltpu.VMEM((4, CK), jnp.int32),
            pltpu.VMEM((CK,), jnp.float32),
            pltpu.VMEM((CK,), jnp.float32),
            pltpu.VMEM((CK, D), jnp.float32),
            pltpu.VMEM((CK, D), jnp.float32),
            pltpu.VMEM_SHARED((N_PAD, D), jnp.float32),
            pltpu.SemaphoreType.DMA,
            pltpu.SemaphoreType.DMA,
        ],
    )


def kernel(h, distances, edges, node_mask, edge_mask, W, Watt, batt, gamma, beta):
    f32 = jnp.float32
    wa = Watt[:, 0:D].astype(f32)
    wb = Watt[:, D:2 * D].astype(f32)
    wc = Watt[0, 2 * D]
    hw, aro, acl = pl.pallas_call(
        _prolog_body,
        out_shape=[
            jax.ShapeDtypeStruct((N, D), f32),
            jax.ShapeDtypeStruct((N, 1), f32),
            jax.ShapeDtypeStruct((N, 1), f32),
        ],
    )(h, W, wa, wb)

    pad = E_PAD - E
    row = jnp.pad(edges[0], (0, pad))
    col = jnp.pad(edges[1], (0, pad))
    dall = jnp.pad(distances[:, 0] * wc + batt[0], (0, pad))
    em = jnp.pad(edge_mask[:, 0], (0, pad))
    pack = jnp.stack([row, col,
                      lax.bitcast_convert_type(dall, jnp.int32),
                      lax.bitcast_convert_type(em, jnp.int32)])
    pack = pack.reshape(4, NCHT, CK).transpose(1, 0, 2)

    out2 = _make_edge_kernel()(hw, aro.reshape(N), acl.reshape(N), pack)

    gp = jnp.concatenate([jnp.zeros((1, 1), f32), gamma.reshape(1, D - 1)], axis=1)
    bp = jnp.concatenate([jnp.zeros((1, 1), f32), beta.reshape(1, D - 1)], axis=1)
    out = pl.pallas_call(
        _epilog_body,
        out_shape=jax.ShapeDtypeStruct((N, D), f32),
    )(out2, gp, bp)
    return (out, distances, edges, node_mask, edge_mask)


# speedup vs baseline: 1.2771x; 1.2771x over previous
import functools

import jax
import jax.numpy as jnp
from jax import lax
from jax.experimental import pallas as pl
from jax.experimental.pallas import tpu as pltpu
from jax.experimental.pallas import tpu_sc as plsc

N = 10000
E = 320000
D = 128
C_CURV = 1.0
K_CURV = 1.0 / C_CURV
SQRTK = K_CURV ** 0.5
EPS = 1e-7
MIN_NORM = 1e-15

NC = 2
NS = 16
CK = 128
N_PAD = 10240
NSLICE = N_PAD // NS
ZROWS = 128
EPT = -(-E // (NC * NS * CK)) * CK
E_PAD = EPT * NC * NS
NCH = EPT // CK
NCHT = NCH * NC * NS


def _prolog_body(h_ref, w_ref, wa_ref, wb_ref, hw_ref, aro_ref, acl_ref):
    h = h_ref[...]
    col = lax.broadcasted_iota(jnp.int32, (N, D), 1)
    is0 = col == 0
    h0 = h[:, 0:1]
    y = jnp.where(is0, 0.0, h)
    y_norm = jnp.maximum(jnp.sqrt(jnp.sum(y * y, axis=1, keepdims=True)), MIN_NORM)
    theta = jnp.maximum(h0 / SQRTK, 1.0 + EPS)
    arcosh = jnp.log(theta + jnp.sqrt(jnp.maximum(theta * theta - 1.0, MIN_NORM)))
    u = jnp.where(is0, 0.0, h * (SQRTK * arcosh / y_norm))
    hw = lax.dot_general(u, w_ref[...], (((1,), (1,)), ((), ())),
                         preferred_element_type=jnp.float32)
    hw_ref[...] = hw
    aro_ref[...] = jnp.sum(hw * wa_ref[...], axis=1, keepdims=True)
    acl_ref[...] = jnp.sum(hw * wb_ref[...], axis=1, keepdims=True)


def _edge_body(hw_hbm, aro_hbm, acl_hbm, pack_hbm, out_hbm, pack_v, ar_v,
               ac_v, att_v, rows_v, srows_v, out_sh, sem, sem2, sem3):
    cid = lax.axis_index("c")
    sid = lax.axis_index("s")
    zv = jnp.zeros((16,), jnp.float32)

    def _zrow(r, _):
        for dd in range(8):
            rows_v[r, pl.ds(dd * 16, 16)] = zv
        return 0

    lax.fori_loop(0, ZROWS, _zrow, 0)
    base = sid * NSLICE
    for j in range(NSLICE // ZROWS):
        pltpu.sync_copy(rows_v, out_sh.at[pl.ds(base + j * ZROWS, ZROWS)])
    plsc.subcore_barrier()

    wid = cid * NS + sid

    dn = lax.GatherDimensionNumbers(offset_dims=(), collapsed_slice_dims=(0,),
                                    start_index_map=(0,))
    H = CK // 2

    def _chunk(c, _):
        cidx = wid * NCH + c
        pltpu.sync_copy(pack_hbm.at[cidx], pack_v)
        da = pltpu.async_copy(hw_hbm.at[pack_v.at[1, pl.ds(0, H)]],
                              rows_v.at[pl.ds(0, H)], sem2)
        db = pltpu.async_copy(hw_hbm.at[pack_v.at[1, pl.ds(H, H)]],
                              rows_v.at[pl.ds(H, H)], sem3)
        d2 = pltpu.async_copy(aro_hbm.at[pack_v.at[0]], ar_v, sem)
        d3 = pltpu.async_copy(acl_hbm.at[pack_v.at[1]], ac_v, sem)
        d2.wait()
        d3.wait()

        def _att(g, _):
            sl = pl.ds(g * 16, 16)
            x = ar_v[sl] + ac_v[sl] + plsc.bitcast(pack_v[2, sl], jnp.float32)
            att_v[sl] = (plsc.bitcast(pack_v[3, sl], jnp.float32)
                         / (1.0 + jnp.exp(-x)))
            return 0

        lax.fori_loop(0, CK // 16, _att, 0)

        def _group(g, _):
            att = att_v[pl.ds(g * 16, 16)]
            ge0 = g * 16
            for e in range(16):
                attb = lax.gather(att, jnp.full((16, 1), e, jnp.int32), dn,
                                  (1,),
                                  mode=lax.GatherScatterMode.PROMISE_IN_BOUNDS)
                for k in range(D // 16):
                    kd = pl.ds(k * 16, 16)
                    srows_v[ge0 + e, kd] = rows_v[ge0 + e, kd] * attb
            return 0

        da.wait()
        lax.fori_loop(0, CK // 32, _group, 0)
        db.wait()
        lax.fori_loop(CK // 32, CK // 16, _group, 0)
        pltpu.sync_copy(srows_v, out_sh.at[pack_v.at[0]], add=True)
        return 0

    lax.fori_loop(0, NCH, _chunk, 0)
    plsc.subcore_barrier()
    pltpu.sync_copy(out_sh.at[pl.ds(base, NSLICE)],
                    out_hbm.at[cid, pl.ds(base, NSLICE)])


def _epilog_body(o2_ref, g_ref, b_ref, out_ref):
    s = (o2_ref[0, 0:N] + o2_ref[1, 0:N]) * 0.01
    col = lax.broadcasted_iota(jnp.int32, (N, D), 1)
    is0 = col == 0
    dm1 = float(D - 1)
    s0 = s[:, 0:1]
    mu = (jnp.sum(s, axis=1, keepdims=True) - s0) / dm1
    dev = jnp.where(is0, 0.0, s - mu)
    var = jnp.sum(dev * dev, axis=1, keepdims=True) / dm1
    tn = dev / jnp.sqrt(var + 1e-5) * g_ref[...] + b_ref[...]
    o = jnp.where(is0, 0.0, tn)
    x_norm = jnp.maximum(jnp.sqrt(jnp.sum(o * o, axis=1, keepdims=True)), MIN_NORM)
    th = x_norm / SQRTK
    e = jnp.exp(th)
    ei = 1.0 / e
    ch = 0.5 * (e + ei)
    sh = 0.5 * (e - ei)
    res = jnp.where(is0, SQRTK * ch, SQRTK * sh * o / x_norm)
    yp = jnp.where(is0, 0.0, res)
    y_sq = jnp.sum(yp * yp, axis=1, keepdims=True)
    r0 = jnp.sqrt(jnp.maximum(K_CURV + y_sq, EPS))
    p = jnp.where(is0, 0.0, SQRTK * res / (r0 + SQRTK))
    p = p / (1.0 + jnp.exp(-p))
    sqn = jnp.sum(p * p, axis=1, keepdims=True)
    inv = SQRTK / (K_CURV - sqn)
    out_ref[...] = jnp.where(is0, (K_CURV + sqn) * inv, (2.0 * SQRTK) * p * inv)


@functools.lru_cache(maxsize=1)
def _make_edge_kernel():
    return pl.kernel(
        _edge_body,
        out_type=jax.ShapeDtypeStruct((NC, N_PAD, D), jnp.float32),
        mesh=plsc.VectorSubcoreMesh(core_axis_name="c", subcore_axis_name="s",
                                    num_cores=NC, num_subcores=NS),
        compiler_params=pltpu.CompilerParams(needs_layout_passes=False),
        scratch_types=[
            pltpu.VMEM((4, CK), jnp.int32),
            pltpu.VMEM((CK,), jnp.float32),
            pltpu.VMEM((CK,), jnp.float32),
            pltpu.VMEM((CK,), jnp.float32),
            pltpu.VMEM((CK, D), jnp.float32),
            pltpu.VMEM((CK, D), jnp.float32),
            pltpu.VMEM_SHARED((N_PAD, D), jnp.float32),
            pltpu.SemaphoreType.DMA,
            pltpu.SemaphoreType.DMA,
            pltpu.SemaphoreType.DMA,
        ],
    )


def kernel(h, distances, edges, node_mask, edge_mask, W, Watt, batt, gamma, beta):
    f32 = jnp.float32
    wa = Watt[:, 0:D].astype(f32)
    wb = Watt[:, D:2 * D].astype(f32)
    wc = Watt[0, 2 * D]
    hw, aro, acl = pl.pallas_call(
        _prolog_body,
        out_shape=[
            jax.ShapeDtypeStruct((N, D), f32),
            jax.ShapeDtypeStruct((N, 1), f32),
            jax.ShapeDtypeStruct((N, 1), f32),
        ],
    )(h, W, wa, wb)

    pad = E_PAD - E
    row = jnp.pad(edges[0], (0, pad))
    col = jnp.pad(edges[1], (0, pad))
    dall = jnp.pad(distances[:, 0] * wc + batt[0], (0, pad))
    em = jnp.pad(edge_mask[:, 0], (0, pad))
    pack = jnp.stack([row, col,
                      lax.bitcast_convert_type(dall, jnp.int32),
                      lax.bitcast_convert_type(em, jnp.int32)])
    pack = pack.reshape(4, NCHT, CK).transpose(1, 0, 2)

    out2 = _make_edge_kernel()(hw, aro.reshape(N), acl.reshape(N), pack)

    gp = jnp.concatenate([jnp.zeros((1, 1), f32), gamma.reshape(1, D - 1)], axis=1)
    bp = jnp.concatenate([jnp.zeros((1, 1), f32), beta.reshape(1, D - 1)], axis=1)
    out = pl.pallas_call(
        _epilog_body,
        out_shape=jax.ShapeDtypeStruct((N, D), f32),
    )(out2, gp, bp)
    return (out, distances, edges, node_mask, edge_mask)
